# baseline (speedup 1.0000x reference)
import functools

import jax
import jax.numpy as jnp
import numpy as np
from jax.experimental import pallas as pl

LATENT = 128
RADIUS = 0.03
NODE_TYPE_SIZE = 9
OBSTACLE = 1
WORLD_EDGE_CAP = 200000


def _edges_from_cells(cells, num_nodes):
    e = jnp.concatenate([cells[:, [0, 1]], cells[:, [1, 2]], cells[:, [2, 0]]], axis=0)
    e = jnp.sort(e, axis=1)
    keys = jnp.sort(e[:, 0] * num_nodes + e[:, 1])
    uniq = jnp.concatenate([jnp.ones((1,), bool), keys[1:] != keys[:-1]])
    e0 = keys // num_nodes
    e1 = keys % num_nodes
    senders = jnp.concatenate([e0, e1])
    receivers = jnp.concatenate([e1, e0])
    seg_mask = jnp.concatenate([uniq, uniq])
    return senders, receivers, seg_mask


def _world_edges_ref(world_pos, senders, receivers, node_type):
    N = world_pos.shape[0]
    obstacle = node_type[:, 0] == OBSTACLE
    mesh_keys = jnp.sort(senders * N + receivers)
    n_keys = mesh_keys.shape[0]
    cols = jnp.arange(N)
    chunk = 500

    def _proc(i0, size, carry):
        ws_b, wr_b, off = carry
        wp_c = jax.lax.dynamic_slice(world_pos, (i0, 0), (size, world_pos.shape[1]))
        rows = i0 + jnp.arange(size)
        diff = wp_c[:, None, :] - world_pos[None, :, :]
        d = jnp.sqrt((diff * diff).sum(-1))
        m = d < RADIUS
        m &= rows[:, None] != cols[None, :]
        obs_c = jax.lax.dynamic_slice(obstacle, (i0,), (size,))
        m &= obs_c[:, None]
        m &= ~obstacle[None, :]
        flat = (rows[:, None] * N + cols[None, :]).ravel()
        pidx = jnp.searchsorted(mesh_keys, flat)
        pidxc = jnp.clip(pidx, 0, n_keys - 1)
        inmesh = (pidx < n_keys) & (mesh_keys[pidxc] == flat)
        mf = m.ravel() & ~inmesh
        pos = off + jnp.cumsum(mf.astype(jnp.int32)) - 1
        pos = jnp.where(mf, pos, WORLD_EDGE_CAP)
        rr = jnp.broadcast_to(rows[:, None], (size, N)).ravel()
        cc = jnp.broadcast_to(cols[None, :], (size, N)).ravel()
        ws_b = ws_b.at[pos].set(rr, mode='drop')
        wr_b = wr_b.at[pos].set(cc, mode='drop')
        off = off + mf.sum(dtype=jnp.int32)
        return ws_b, wr_b, off

    ws0 = jnp.full((WORLD_EDGE_CAP,), N, jnp.int32)
    wr0 = jnp.full((WORLD_EDGE_CAP,), N, jnp.int32)
    carry = (ws0, wr0, jnp.int32(0))

    def body(i, carry):
        return _proc(i * chunk, chunk, carry)

    carry = jax.lax.fori_loop(0, N // chunk, body, carry)
    ws, wr, _ = carry
    return ws, wr


def _round_up(x, m):
    return (x + m - 1) // m * m


def _mlp_body(x_ref, w1_ref, b1_ref, w2_ref, b2_ref, o_ref):
    h = jnp.maximum(
        jnp.dot(x_ref[...], w1_ref[...], preferred_element_type=jnp.float32)
        + b1_ref[...], 0.0)
    o_ref[...] = (jnp.dot(h, w2_ref[...], preferred_element_type=jnp.float32)
                  + b2_ref[...])


def _mlp_res_body(x_ref, w1_ref, b1_ref, w2_ref, b2_ref, r_ref, o_ref):
    h = jnp.maximum(
        jnp.dot(x_ref[...], w1_ref[...], preferred_element_type=jnp.float32)
        + b1_ref[...], 0.0)
    o_ref[...] = (jnp.dot(h, w2_ref[...], preferred_element_type=jnp.float32)
                  + b2_ref[...] + r_ref[...])


def _mlp_pallas(x, p, res=None, block_rows=2048):
    R, Din = x.shape
    W1, b1, W2, b2 = p['W1'], p['b1'], p['W2'], p['b2']
    Dh = W1.shape[1]
    Dout = W2.shape[1]
    Kp = _round_up(Din, 128)
    Hp = _round_up(Dh, 128)
    Op = _round_up(Dout, 128)
    Rp = _round_up(R, block_rows)
    xp = jnp.zeros((Rp, Kp), x.dtype).at[:R, :Din].set(x)
    W1p = jnp.zeros((Kp, Hp), W1.dtype).at[:Din, :Dh].set(W1)
    b1p = jnp.zeros((1, Hp), b1.dtype).at[0, :Dh].set(b1)
    W2p = jnp.zeros((Hp, Op), W2.dtype).at[:Dh, :Dout].set(W2)
    b2p = jnp.zeros((1, Op), b2.dtype).at[0, :Dout].set(b2)
    grid = (Rp // block_rows,)
    in_specs = [
        pl.BlockSpec((block_rows, Kp), lambda i: (i, 0)),
        pl.BlockSpec((Kp, Hp), lambda i: (0, 0)),
        pl.BlockSpec((1, Hp), lambda i: (0, 0)),
        pl.BlockSpec((Hp, Op), lambda i: (0, 0)),
        pl.BlockSpec((1, Op), lambda i: (0, 0)),
    ]
    args = [xp, W1p, b1p, W2p, b2p]
    body = _mlp_body
    if res is not None:
        resp = jnp.zeros((Rp, Op), res.dtype).at[:R, :Dout].set(res)
        in_specs.append(pl.BlockSpec((block_rows, Op), lambda i: (i, 0)))
        args.append(resp)
        body = _mlp_res_body
    out = pl.pallas_call(
        body,
        grid=grid,
        in_specs=in_specs,
        out_specs=pl.BlockSpec((block_rows, Op), lambda i: (i, 0)),
        out_shape=jax.ShapeDtypeStruct((Rp, Op), jnp.float32),
    )(*args)
    return out[:R, :Dout]


def _forward(world_pos, mesh_pos, params, node_type, s, r, ws, wr, r_seg):
    N = world_pos.shape[0]
    one_hot = jax.nn.one_hot(node_type[:, 0], NODE_TYPE_SIZE, dtype=jnp.float32)
    node_feat = (one_hot - params['node_mean']) / params['node_std']
    rel_mesh = mesh_pos[s] - mesh_pos[r]
    rel_world_m = world_pos[s] - world_pos[r]
    mesh_feat = jnp.concatenate([
        rel_mesh, jnp.linalg.norm(rel_mesh, axis=-1, keepdims=True),
        rel_world_m, jnp.linalg.norm(rel_world_m, axis=-1, keepdims=True)], axis=-1)
    mesh_feat = (mesh_feat - params['mesh_e_mean']) / params['mesh_e_std']
    wsc = jnp.minimum(ws, N - 1)
    wrc = jnp.minimum(wr, N - 1)
    rel_world = world_pos[wrc] - world_pos[wsc]
    world_feat = jnp.concatenate([
        rel_world, jnp.linalg.norm(rel_world, axis=-1, keepdims=True)], axis=-1)
    world_feat = (world_feat - params['world_e_mean']) / params['world_e_std']
    node_lat = _mlp_pallas(node_feat, params['node_enc'])
    mesh_lat = _mlp_pallas(mesh_feat, params['mesh_enc'])
    world_lat = _mlp_pallas(world_feat, params['world_enc'])
    for blk in params['mp']:
        new_mesh = _mlp_pallas(
            jnp.concatenate([mesh_lat, node_lat[s], node_lat[r]], -1),
            blk['mesh'], res=mesh_lat)
        new_world = _mlp_pallas(
            jnp.concatenate([world_lat, node_lat[wsc], node_lat[wrc]], -1),
            blk['world'], res=world_lat)
        agg_mesh = jax.ops.segment_sum(new_mesh, r_seg, num_segments=N)
        agg_world = jax.ops.segment_sum(new_world, wr, num_segments=N)
        node_lat = _mlp_pallas(
            jnp.concatenate([node_lat, agg_mesh, agg_world], -1),
            blk['node'], res=node_lat)
        mesh_lat, world_lat = new_mesh, new_world
    out = _mlp_pallas(node_lat, params['dec'])
    return out * params['out_std'] + params['out_mean']


def kernel(world_pos, mesh_pos, node_type, cells, params):
    N = world_pos.shape[0]
    senders, receivers, seg_mask = _edges_from_cells(cells, N)
    ws, wr = _world_edges_ref(world_pos, senders, receivers, node_type)
    r_seg = jnp.where(seg_mask, receivers, N)
    return _forward(world_pos, mesh_pos, params, node_type,
                    senders, receivers, ws, wr, r_seg)
